# Initial kernel scaffold; baseline (speedup 1.0000x reference)
import jax
import jax.numpy as jnp
from jax.experimental import pallas as pl


def kernel(team_ID, player_ids, season_ID, down_ID, W_team, W_player, W_season, W_down):
    raise NotImplementedError("write your pallas kernel here")


# speedup vs baseline: 2.0191x; 2.0191x over previous
import functools

import jax
import jax.numpy as jnp
from jax import lax
from jax.experimental import pallas as pl
from jax.experimental.pallas import tpu as pltpu
from jax.experimental.pallas import tpu_sc as plsc

BATCH = 16384
D = 64
NC = 2
NS = 16
NW = NC * NS
B_PER_W = BATCH // NW
CHUNK = 128
N_CHUNKS = B_PER_W // CHUNK


def _body(w_team, w_player, w_season, w_down,
          idx_team, idx_player, idx_season, idx_down,
          out,
          iv_t, iv_p, iv_s, iv_d,
          b_t, b_p, b_s, b_d, o_v,
          sem):
    wid = lax.axis_index("s") * NC + lax.axis_index("c")
    base = wid * B_PER_W

    pltpu.sync_copy(idx_team.at[wid], iv_t)
    pltpu.sync_copy(idx_player.at[wid], iv_p)
    pltpu.sync_copy(idx_season.at[wid], iv_s)
    pltpu.sync_copy(idx_down.at[wid], iv_d)

    for c in range(N_CHUNKS):
        h_t = pltpu.async_copy(w_team.at[iv_t.at[c]], b_t, sem)
        h_p = pltpu.async_copy(w_player.at[iv_p.at[c]], b_p, sem)
        h_s = pltpu.async_copy(w_season.at[iv_s.at[c]], b_s, sem)
        h_d = pltpu.async_copy(w_down.at[iv_d.at[c]], b_d, sem)
        h_t.wait()
        h_p.wait()
        h_s.wait()
        h_d.wait()

        def row(r, _):
            for j in range(D // 16):
                sl = pl.ds(j * 16, 16)
                o_v[r, sl] = (b_t[r, sl] + b_p[r, sl]) + (b_s[r, sl] + b_d[r, sl])
            return 0

        lax.fori_loop(0, CHUNK, row, 0)
        pltpu.sync_copy(o_v, out.at[pl.ds(base + c * CHUNK, CHUNK)])


@jax.jit
def _meta_embed(team_ID, player_ids, season_ID, down_ID,
                W_team, W_player, W_season, W_down):
    it = jnp.reshape(team_ID.astype(jnp.int32), (NW, N_CHUNKS, CHUNK))
    ip = jnp.reshape(player_ids.astype(jnp.int32), (NW, N_CHUNKS, CHUNK))
    isn = jnp.reshape(season_ID.astype(jnp.int32), (NW, N_CHUNKS, CHUNK))
    idn = jnp.reshape(down_ID.astype(jnp.int32), (NW, N_CHUNKS, CHUNK))

    run = pl.kernel(
        _body,
        out_type=jax.ShapeDtypeStruct((BATCH, D), jnp.float32),
        mesh=plsc.VectorSubcoreMesh(
            core_axis_name="c", subcore_axis_name="s",
            num_cores=NC, num_subcores=NS),
        scratch_types=[
            pltpu.VMEM((N_CHUNKS, CHUNK), jnp.int32),
            pltpu.VMEM((N_CHUNKS, CHUNK), jnp.int32),
            pltpu.VMEM((N_CHUNKS, CHUNK), jnp.int32),
            pltpu.VMEM((N_CHUNKS, CHUNK), jnp.int32),
            pltpu.VMEM((CHUNK, D), jnp.float32),
            pltpu.VMEM((CHUNK, D), jnp.float32),
            pltpu.VMEM((CHUNK, D), jnp.float32),
            pltpu.VMEM((CHUNK, D), jnp.float32),
            pltpu.VMEM((CHUNK, D), jnp.float32),
            pltpu.SemaphoreType.DMA,
        ],
        compiler_params=pltpu.CompilerParams(use_tc_tiling_on_sc=False),
    )
    return run(W_team, W_player, W_season, W_down, it, ip, isn, idn)


def kernel(team_ID, player_ids, season_ID, down_ID,
           W_team, W_player, W_season, W_down):
    return _meta_embed(team_ID, player_ids, season_ID, down_ID,
                       W_team, W_player, W_season, W_down)


# speedup vs baseline: 2.0197x; 1.0003x over previous
import jax
import jax.numpy as jnp
from jax import lax
from jax.experimental import pallas as pl
from jax.experimental.pallas import tpu as pltpu
from jax.experimental.pallas import tpu_sc as plsc

BATCH = 16384
D = 64
NC = 2
NS = 16
NW = NC * NS
B_PER_W = BATCH // NW
CHUNK = 128
N_CHUNKS = B_PER_W // CHUNK


def _body(w_team, w_player, w_season, w_down,
          idx_team, idx_player, idx_season, idx_down,
          out,
          iv_t, iv_p, iv_s, iv_d,
          b_t, b_p, b_s, b_d, o_v,
          sem):
    wid = lax.axis_index("s") * NC + lax.axis_index("c")
    base = wid * B_PER_W

    pltpu.sync_copy(idx_team.at[pl.ds(base, B_PER_W)], iv_t)
    pltpu.sync_copy(idx_player.at[pl.ds(base, B_PER_W)], iv_p)
    pltpu.sync_copy(idx_season.at[pl.ds(base, B_PER_W)], iv_s)
    pltpu.sync_copy(idx_down.at[pl.ds(base, B_PER_W)], iv_d)

    for c in range(N_CHUNKS):
        sl_idx = pl.ds(c * CHUNK, CHUNK)
        h_t = pltpu.async_copy(w_team.at[iv_t.at[sl_idx]], b_t, sem)
        h_p = pltpu.async_copy(w_player.at[iv_p.at[sl_idx]], b_p, sem)
        h_s = pltpu.async_copy(w_season.at[iv_s.at[sl_idx]], b_s, sem)
        h_d = pltpu.async_copy(w_down.at[iv_d.at[sl_idx]], b_d, sem)
        h_t.wait()
        h_p.wait()
        h_s.wait()
        h_d.wait()

        def row(r, _):
            for j in range(D // 16):
                sl = pl.ds(j * 16, 16)
                o_v[r, sl] = (b_t[r, sl] + b_p[r, sl]) + (b_s[r, sl] + b_d[r, sl])
            return 0

        lax.fori_loop(0, CHUNK, row, 0)
        pltpu.sync_copy(o_v, out.at[pl.ds(base + c * CHUNK, CHUNK)])


@jax.jit
def _meta_embed(team_ID, player_ids, season_ID, down_ID,
                W_team, W_player, W_season, W_down):
    run = pl.kernel(
        _body,
        out_type=jax.ShapeDtypeStruct((BATCH, D), jnp.float32),
        mesh=plsc.VectorSubcoreMesh(
            core_axis_name="c", subcore_axis_name="s",
            num_cores=NC, num_subcores=NS),
        scratch_types=[
            pltpu.VMEM((B_PER_W,), jnp.int32),
            pltpu.VMEM((B_PER_W,), jnp.int32),
            pltpu.VMEM((B_PER_W,), jnp.int32),
            pltpu.VMEM((B_PER_W,), jnp.int32),
            pltpu.VMEM((CHUNK, D), jnp.float32),
            pltpu.VMEM((CHUNK, D), jnp.float32),
            pltpu.VMEM((CHUNK, D), jnp.float32),
            pltpu.VMEM((CHUNK, D), jnp.float32),
            pltpu.VMEM((CHUNK, D), jnp.float32),
            pltpu.SemaphoreType.DMA,
        ],
        compiler_params=pltpu.CompilerParams(use_tc_tiling_on_sc=False),
    )
    return run(W_team, W_player, W_season, W_down,
               team_ID.astype(jnp.int32), player_ids.astype(jnp.int32),
               season_ID.astype(jnp.int32), down_ID.astype(jnp.int32))


def kernel(team_ID, player_ids, season_ID, down_ID,
           W_team, W_player, W_season, W_down):
    return _meta_embed(team_ID, player_ids, season_ID, down_ID,
                       W_team, W_player, W_season, W_down)
